# baseline (speedup 1.0000x reference)
import functools

import jax
import jax.numpy as jnp
from jax import lax
from jax.experimental import pallas as pl
from jax.experimental.pallas import tpu as pltpu
from jax.experimental.pallas import tpu_sc as plsc

E = 8
K = 2
D = 1024
F = 4096
FB = F // 2
TM = 128
RB = 1024
LANES = 128

NC = 2
NS = 16
NW = NC * NS
CH = 32

_mesh = plsc.VectorSubcoreMesh(core_axis_name="c", subcore_axis_name="s")


def _router_body(x_ref, gw_ref, s_ref, e_ref, w_ref, c_ref):
    @pl.when(pl.program_id(0) == 0)
    def _():
        c_ref[...] = jnp.zeros_like(c_ref)

    x = x_ref[...]
    gw = gw_ref[...]
    logits = jnp.dot(x, gw, preferred_element_type=jnp.float32)
    lane = lax.broadcasted_iota(jnp.int32, logits.shape, 1)
    valid = lane < E
    neg = jnp.float32(-1e30)
    l = jnp.where(valid, logits, neg)
    m = jnp.max(l, axis=1, keepdims=True)
    p = jnp.where(valid, jnp.exp(l - m), 0.0)
    s = p / jnp.sum(p, axis=1, keepdims=True)
    big = jnp.int32(LANES * 2)
    m1 = jnp.max(s, axis=1, keepdims=True)
    e1 = jnp.min(jnp.where((s == m1) & valid, lane, big), axis=1, keepdims=True)
    s_wo1 = jnp.where(lane == e1, -1.0, jnp.where(valid, s, -1.0))
    m2 = jnp.max(s_wo1, axis=1, keepdims=True)
    e2 = jnp.min(jnp.where((s_wo1 == m2) & valid, lane, big), axis=1, keepdims=True)
    s_ref[...] = jnp.where(lane == 0, m1, jnp.where(lane == 1, m2, 0.0))
    e_ref[...] = jnp.where(lane == 0, e1, jnp.where(lane == 1, e2, 0))
    oh1 = (lane == e1).astype(jnp.bfloat16)
    oh2 = (lane == e2).astype(jnp.bfloat16)
    ohs = oh1 + oh2
    r_i = lax.broadcasted_iota(jnp.int32, (RB, RB), 0)
    c_i = lax.broadcasted_iota(jnp.int32, (RB, RB), 1)
    lt = (r_i > c_i).astype(jnp.bfloat16)
    pref = jnp.dot(lt, ohs, preferred_element_type=jnp.float32)
    base = c_ref[...].astype(jnp.float32)
    tot = pref + base
    w0 = jnp.sum(tot * oh1.astype(jnp.float32), axis=1, keepdims=True)
    w1 = jnp.sum(tot * oh2.astype(jnp.float32), axis=1, keepdims=True)
    w_ref[...] = jnp.where(lane == 0, w0, jnp.where(lane == 1, w1, 0.0)).astype(jnp.int32)
    c_ref[...] = (base + jnp.sum(ohs.astype(jnp.float32), axis=0, keepdims=True)).astype(jnp.int32)


def _router(x_flat, gw_pad, n):
    return pl.pallas_call(
        _router_body,
        grid=(n // RB,),
        in_specs=[
            pl.BlockSpec((RB, D), lambda i: (i, 0)),
            pl.BlockSpec((D, LANES), lambda i: (0, 0)),
        ],
        out_specs=[
            pl.BlockSpec((RB, LANES), lambda i: (i, 0)),
            pl.BlockSpec((RB, LANES), lambda i: (i, 0)),
            pl.BlockSpec((RB, LANES), lambda i: (i, 0)),
            pl.BlockSpec((1, LANES), lambda i: (0, 0)),
        ],
        out_shape=[
            jax.ShapeDtypeStruct((n, LANES), jnp.float32),
            jax.ShapeDtypeStruct((n, LANES), jnp.int32),
            jax.ShapeDtypeStruct((n, LANES), jnp.int32),
            jax.ShapeDtypeStruct((1, LANES), jnp.int32),
        ],
    )(x_flat, gw_pad)


def _make_sc_permute(s_slots, p_rows):
    slots_w = s_slots // NW
    nch = slots_w // CH

    @functools.partial(
        pl.kernel,
        out_type=[
            jax.ShapeDtypeStruct((p_rows, D), jnp.float32),
            jax.ShapeDtypeStruct((p_rows, LANES), jnp.float32),
        ],
        mesh=_mesh,
        scratch_types=[
            pltpu.VMEM((nch, CH), jnp.int32),
            pltpu.VMEM((slots_w,), jnp.int32),
            pltpu.VMEM((CH, D), jnp.float32),
            pltpu.VMEM((CH, D), jnp.float32),
            pltpu.VMEM((CH, LANES), jnp.float32),
            pltpu.VMEM((CH, LANES), jnp.float32),
        ]
        + [pltpu.SemaphoreType.DMA] * 8,
    )
    def permute(x_hbm, rank_hbm, tok_hbm, sb_hbm, xg_hbm, sp_hbm,
                idx_v, tok_v, r0, r1, q0, q1,
                g0, g1, h0, h1, ox0, ox1, os0, os1):
        wid = lax.axis_index("s") * NC + lax.axis_index("c")
        j0 = wid * slots_w
        pltpu.sync_copy(rank_hbm.at[wid], idx_v)
        pltpu.sync_copy(tok_hbm.at[pl.ds(j0, slots_w)], tok_v)
        rbuf = (r0, r1)
        qbuf = (q0, q1)
        gsem = (g0, g1)
        hsem = (h0, h1)
        oxsem = (ox0, ox1)
        ossem = (os0, os1)

        def start_in(c, b):
            gh = pltpu.async_copy(
                x_hbm.at[tok_v.at[pl.ds(c * CH, CH)]], rbuf[b], gsem[b])
            sh = pltpu.async_copy(
                sb_hbm.at[pl.ds(j0 + c * CH, CH)], qbuf[b], hsem[b])
            return gh, sh

        pend = [start_in(0, 0), start_in(1, 1)]
        out_pend = [None, None]
        for c in range(nch):
            b = c % 2
            gh, sh = pend[b]
            gh.wait()
            sh.wait()
            oh = (
                pltpu.async_copy(rbuf[b], xg_hbm.at[idx_v.at[c]], oxsem[b]),
                pltpu.async_copy(qbuf[b], sp_hbm.at[idx_v.at[c]], ossem[b]),
            )
            out_pend[b] = oh
            if c + 2 < nch:
                oh[0].wait()
                oh[1].wait()
                out_pend[b] = None
                pend[b] = start_in(c + 2, b)
        for b in range(2):
            if out_pend[b] is not None:
                out_pend[b][0].wait()
                out_pend[b][1].wait()

    return permute


def _run_schedule(key_flat, src_a, src_b):
    L = key_flat.shape[0]
    first = jnp.concatenate([
        jnp.ones((1,), jnp.int32),
        (key_flat[1:] != key_flat[:-1]).astype(jnp.int32),
    ])
    run_id = jnp.cumsum(first) - 1
    rr = jnp.arange(L, dtype=jnp.int32)[:, None]
    mask = (run_id[None, :] == rr) & (first[None, :] == 1)
    def per_run(v):
        r = jnp.max(jnp.where(mask, v[None, :], -1), axis=1)
        return jnp.concatenate([r, jnp.full((1,), -1, jnp.int32)])
    a_by_run = per_run(src_a)
    b_by_run = per_run(src_b)
    nxt = jnp.clip(run_id + 1, 0, L)
    pf_a = jnp.where(first == 1, a_by_run[nxt], -1).astype(jnp.int32)
    pf_b = jnp.where(first == 1, b_by_run[nxt], -1).astype(jnp.int32)
    pf_slot = ((run_id + 1) % 2).astype(jnp.int32)
    cur_slot = (run_id % 2).astype(jnp.int32)
    k0 = (jnp.arange(L, dtype=jnp.int32) == 0)
    i2_a = jnp.where(k0, src_a[0], -1).astype(jnp.int32)
    i2_b = jnp.where(k0, src_b[0], -1).astype(jnp.int32)
    return first, cur_slot, pf_a, pf_b, pf_slot, i2_a, i2_b


def _ffn_a_body(first_r, cs_r, pfe_r, pfj_r, pfs_r, i2e_r, i2j_r,
                xg_ref, s_ref, wg_hbm, wu_hbm, h_ref,
                wgb, wub, sg, su):
    t_steps = pl.num_programs(1)
    k = pl.program_id(0) * t_steps + pl.program_id(1)
    e2 = i2e_r[k]
    j2 = i2j_r[k]

    @pl.when(e2 >= 0)
    def _():
        sl2 = pl.ds(j2 * FB, FB)
        pltpu.make_async_copy(wg_hbm.at[e2, :, sl2], wgb.at[0], sg.at[0]).start()
        pltpu.make_async_copy(wu_hbm.at[e2, :, sl2], wub.at[0], su.at[0]).start()

    pe = pfe_r[k]
    pj = pfj_r[k]
    ps = pfs_r[k]

    @pl.when(pe >= 0)
    def _():
        slp = pl.ds(pj * FB, FB)
        pltpu.make_async_copy(wg_hbm.at[pe, :, slp], wgb.at[ps], sg.at[ps]).start()
        pltpu.make_async_copy(wu_hbm.at[pe, :, slp], wub.at[ps], su.at[ps]).start()

    cs = cs_r[k]

    @pl.when(first_r[k] == 1)
    def _():
        sl0 = pl.ds(0, FB)
        pltpu.make_async_copy(wg_hbm.at[0, :, sl0], wgb.at[cs], sg.at[cs]).wait()
        pltpu.make_async_copy(wu_hbm.at[0, :, sl0], wub.at[cs], su.at[cs]).wait()

    xs = (xg_ref[...] * s_ref[:, 0:1]).astype(jnp.bfloat16)
    wg = wgb[cs].astype(jnp.bfloat16)
    wu = wub[cs].astype(jnp.bfloat16)
    g = jnp.dot(xs, wg, preferred_element_type=jnp.float32)
    u = jnp.dot(xs, wu, preferred_element_type=jnp.float32)
    h_ref[...] = (g * (1.0 / (1.0 + jnp.exp(-g))) * u).astype(jnp.bfloat16)


def _ffn_b_body(first_r, cs_r, pfe_r, pfs_r, i2e_r,
                h_ref, wd_hbm, out_ref, wdb, sd):
    k = pl.program_id(0)
    e2 = i2e_r[k]

    @pl.when(e2 >= 0)
    def _():
        pltpu.make_async_copy(wd_hbm.at[e2], wdb.at[0], sd.at[0]).start()

    pe = pfe_r[k]
    ps = pfs_r[k]

    @pl.when(pe >= 0)
    def _():
        pltpu.make_async_copy(wd_hbm.at[pe], wdb.at[ps], sd.at[ps]).start()

    cs = cs_r[k]

    @pl.when(first_r[k] == 1)
    def _():
        pltpu.make_async_copy(wd_hbm.at[0], wdb.at[cs], sd.at[cs]).wait()

    h = h_ref[...]
    wd = wdb[cs].astype(jnp.bfloat16)
    out_ref[...] = jnp.dot(h, wd, preferred_element_type=jnp.float32)


def _ffn(te, xg, s_p, w_gate, w_up, w_down, p_rows):
    t = p_rows // TM
    nj = F // FB
    jj = jnp.repeat(jnp.arange(nj, dtype=jnp.int32), t)
    ee = jnp.tile(te, nj)
    key = ee * nj + jj
    fa, csa, pfe, pfj, pfs, i2e, i2j = _run_schedule(key, ee, jj)
    h = pl.pallas_call(
        _ffn_a_body,
        grid_spec=pltpu.PrefetchScalarGridSpec(
            num_scalar_prefetch=7,
            grid=(nj, t),
            in_specs=[
                pl.BlockSpec((TM, D), lambda j, i, *_: (i, 0)),
                pl.BlockSpec((TM, LANES), lambda j, i, *_: (i, 0)),
                pl.BlockSpec(memory_space=pl.ANY),
                pl.BlockSpec(memory_space=pl.ANY),
            ],
            out_specs=pl.BlockSpec((TM, FB), lambda j, i, *_: (i, j)),
            scratch_shapes=[
                pltpu.VMEM((2, D, FB), jnp.float32),
                pltpu.VMEM((2, D, FB), jnp.float32),
                pltpu.SemaphoreType.DMA((2,)),
                pltpu.SemaphoreType.DMA((2,)),
            ],
        ),
        out_shape=jax.ShapeDtypeStruct((p_rows, F), jnp.bfloat16),
    )(fa, csa, pfe, pfj, pfs, i2e, i2j, xg, s_p, w_gate, w_up)

    fb, csb, pfe_b, _unused, pfs_b, i2e_b, _u2 = _run_schedule(
        te, te, jnp.zeros_like(te))
    return pl.pallas_call(
        _ffn_b_body,
        grid_spec=pltpu.PrefetchScalarGridSpec(
            num_scalar_prefetch=5,
            grid=(t,),
            in_specs=[
                pl.BlockSpec((TM, F), lambda i, *_: (i, 0)),
                pl.BlockSpec(memory_space=pl.ANY),
            ],
            out_specs=pl.BlockSpec((TM, D), lambda i, *_: (i, 0)),
            scratch_shapes=[
                pltpu.VMEM((2, F, D), jnp.float32),
                pltpu.SemaphoreType.DMA((2,)),
            ],
        ),
        out_shape=jax.ShapeDtypeStruct((p_rows, D), jnp.float32),
    )(fb, csb, pfe_b, pfs_b, i2e_b, h, w_down)


def _make_sc_combine(n, cc):
    tok_w = n // NW
    nch = tok_w // cc

    @functools.partial(
        pl.kernel,
        out_type=jax.ShapeDtypeStruct((n, D), jnp.float32),
        mesh=_mesh,
        scratch_types=[
            pltpu.VMEM((tok_w,), jnp.int32),
            pltpu.VMEM((tok_w,), jnp.int32),
            pltpu.VMEM((cc, D), jnp.float32),
            pltpu.VMEM((cc, D), jnp.float32),
            pltpu.VMEM((cc, D), jnp.float32),
            pltpu.VMEM((cc, D), jnp.float32),
        ]
        + [pltpu.SemaphoreType.DMA] * 6,
    )
    def combine(y_hbm, c0_hbm, c1_hbm, out_hbm, i0_v, i1_v, a0, a1, b0, b1,
                ga0, ga1, gb0, gb1, st0, st1):
        wid = lax.axis_index("s") * NC + lax.axis_index("c")
        base = wid * tok_w
        pltpu.sync_copy(c0_hbm.at[pl.ds(base, tok_w)], i0_v)
        pltpu.sync_copy(c1_hbm.at[pl.ds(base, tok_w)], i1_v)
        abuf = (a0, a1)
        bbuf = (b0, b1)
        gas = (ga0, ga1)
        gbs = (gb0, gb1)
        sts = (st0, st1)

        def start_in(c, b):
            return (
                pltpu.async_copy(y_hbm.at[i0_v.at[pl.ds(c * cc, cc)]], abuf[b], gas[b]),
                pltpu.async_copy(y_hbm.at[i1_v.at[pl.ds(c * cc, cc)]], bbuf[b], gbs[b]),
            )

        pend = [start_in(0, 0), start_in(1, 1)]
        out_pend = [None, None]
        for c in range(nch):
            b = c % 2
            pend[b][0].wait()
            pend[b][1].wait()
            a_v, b_v = abuf[b], bbuf[b]

            def add_row(r, carry):
                for k in range(D // 16):
                    sl = pl.ds(k * 16, 16)
                    a_v[r, sl] = a_v[r, sl] + b_v[r, sl]
                return carry

            lax.fori_loop(0, cc, add_row, 0)
            oh = pltpu.async_copy(a_v, out_hbm.at[pl.ds(base + c * cc, cc)], sts[b])
            out_pend[b] = oh
            if c + 2 < nch:
                oh.wait()
                out_pend[b] = None
                pend[b] = start_in(c + 2, b)
        for b in range(2):
            if out_pend[b] is not None:
                out_pend[b].wait()

    return combine


def kernel(x, gate_w, w_gate, w_up, w_down):
    bs, slen, d = x.shape
    n = bs * slen
    s_slots = n * K
    p_rows = s_slots + E * TM
    x_flat = x.reshape(n, d)

    gw_pad = jnp.zeros((d, LANES), jnp.float32).at[:, :E].set(gate_w)
    srt, idt, wnt, cnt = _router(x_flat, gw_pad, n)
    sco = srt[:, :K].reshape(-1)
    sel = idt[:, :K].reshape(-1)
    within = wnt[:, :K].reshape(-1)

    counts = cnt[0, :E]
    pc = ((counts + TM - 1) // TM) * TM
    ends = jnp.cumsum(pc)
    starts = ends - pc
    rank = (starts[sel] + within).astype(jnp.int32)
    tok = (jnp.arange(s_slots, dtype=jnp.int32) // K)
    tile_start = jnp.arange(p_rows // TM, dtype=jnp.int32) * TM
    te = jnp.sum((tile_start[:, None] >= ends[None, :]).astype(jnp.int32), axis=1)
    te = jnp.clip(te, 0, E - 1).astype(jnp.int32)

    rank3 = rank.reshape(NW, (s_slots // NW) // CH, CH)
    sb = jnp.broadcast_to(sco[:, None], (s_slots, LANES))
    xg, s_p = _make_sc_permute(s_slots, p_rows)(x_flat, rank3, tok, sb)

    y = _ffn(te, xg, s_p, w_gate, w_up, w_down, p_rows)

    cidx = rank.reshape(n, K)
    out = _make_sc_combine(n, 16)(y, cidx[:, 0], cidx[:, 1])
    return out.reshape(bs, slen, d)


# speedup vs baseline: 1.0029x; 1.0029x over previous
import functools

import jax
import jax.numpy as jnp
from jax import lax
from jax.experimental import pallas as pl
from jax.experimental.pallas import tpu as pltpu
from jax.experimental.pallas import tpu_sc as plsc

E = 8
K = 2
D = 1024
F = 4096
FB = F // 2
TM = 128
RB = 512
LANES = 128

NC = 2
NS = 16
NW = NC * NS
CH = 32

_mesh = plsc.VectorSubcoreMesh(core_axis_name="c", subcore_axis_name="s")


def _router_body(x_ref, gw_ref, s_ref, e_ref, w_ref, c_ref):
    @pl.when(pl.program_id(0) == 0)
    def _():
        c_ref[...] = jnp.zeros_like(c_ref)

    x = x_ref[...]
    gw = gw_ref[...]
    logits = jnp.dot(x, gw, preferred_element_type=jnp.float32)
    lane = lax.broadcasted_iota(jnp.int32, logits.shape, 1)
    valid = lane < E
    neg = jnp.float32(-1e30)
    l = jnp.where(valid, logits, neg)
    m = jnp.max(l, axis=1, keepdims=True)
    p = jnp.where(valid, jnp.exp(l - m), 0.0)
    s = p / jnp.sum(p, axis=1, keepdims=True)
    big = jnp.int32(LANES * 2)
    m1 = jnp.max(s, axis=1, keepdims=True)
    e1 = jnp.min(jnp.where((s == m1) & valid, lane, big), axis=1, keepdims=True)
    s_wo1 = jnp.where(lane == e1, -1.0, jnp.where(valid, s, -1.0))
    m2 = jnp.max(s_wo1, axis=1, keepdims=True)
    e2 = jnp.min(jnp.where((s_wo1 == m2) & valid, lane, big), axis=1, keepdims=True)
    s_ref[...] = jnp.where(lane == 0, m1, jnp.where(lane == 1, m2, 0.0))
    e_ref[...] = jnp.where(lane == 0, e1, jnp.where(lane == 1, e2, 0))
    oh1 = (lane == e1).astype(jnp.bfloat16)
    oh2 = (lane == e2).astype(jnp.bfloat16)
    ohs = oh1 + oh2
    r_i = lax.broadcasted_iota(jnp.int32, (RB, RB), 0)
    c_i = lax.broadcasted_iota(jnp.int32, (RB, RB), 1)
    lt = (r_i > c_i).astype(jnp.bfloat16)
    pref = jnp.dot(lt, ohs, preferred_element_type=jnp.float32)
    base = c_ref[...].astype(jnp.float32)
    tot = pref + base
    w0 = jnp.sum(tot * oh1.astype(jnp.float32), axis=1, keepdims=True)
    w1 = jnp.sum(tot * oh2.astype(jnp.float32), axis=1, keepdims=True)
    w_ref[...] = jnp.where(lane == 0, w0, jnp.where(lane == 1, w1, 0.0)).astype(jnp.int32)
    c_ref[...] = (base + jnp.sum(ohs.astype(jnp.float32), axis=0, keepdims=True)).astype(jnp.int32)


def _router(x_flat, gw_pad, n):
    return pl.pallas_call(
        _router_body,
        grid=(n // RB,),
        in_specs=[
            pl.BlockSpec((RB, D), lambda i: (i, 0)),
            pl.BlockSpec((D, LANES), lambda i: (0, 0)),
        ],
        out_specs=[
            pl.BlockSpec((RB, LANES), lambda i: (i, 0)),
            pl.BlockSpec((RB, LANES), lambda i: (i, 0)),
            pl.BlockSpec((RB, LANES), lambda i: (i, 0)),
            pl.BlockSpec((1, LANES), lambda i: (0, 0)),
        ],
        out_shape=[
            jax.ShapeDtypeStruct((n, LANES), jnp.float32),
            jax.ShapeDtypeStruct((n, LANES), jnp.int32),
            jax.ShapeDtypeStruct((n, LANES), jnp.int32),
            jax.ShapeDtypeStruct((1, LANES), jnp.int32),
        ],
    )(x_flat, gw_pad)


def _make_sc_permute(s_slots, p_rows):
    slots_w = s_slots // NW
    nch = slots_w // CH

    @functools.partial(
        pl.kernel,
        out_type=[
            jax.ShapeDtypeStruct((p_rows, D), jnp.float32),
            jax.ShapeDtypeStruct((p_rows, LANES), jnp.float32),
        ],
        mesh=_mesh,
        scratch_types=[
            pltpu.VMEM((nch, CH), jnp.int32),
            pltpu.VMEM((slots_w,), jnp.int32),
            pltpu.VMEM((CH, D), jnp.float32),
            pltpu.VMEM((CH, D), jnp.float32),
            pltpu.VMEM((CH, LANES), jnp.float32),
            pltpu.VMEM((CH, LANES), jnp.float32),
        ]
        + [pltpu.SemaphoreType.DMA] * 8,
    )
    def permute(x_hbm, rank_hbm, tok_hbm, sb_hbm, xg_hbm, sp_hbm,
                idx_v, tok_v, r0, r1, q0, q1,
                g0, g1, h0, h1, ox0, ox1, os0, os1):
        wid = lax.axis_index("s") * NC + lax.axis_index("c")
        j0 = wid * slots_w
        pltpu.sync_copy(rank_hbm.at[wid], idx_v)
        pltpu.sync_copy(tok_hbm.at[pl.ds(j0, slots_w)], tok_v)
        rbuf = (r0, r1)
        qbuf = (q0, q1)
        gsem = (g0, g1)
        hsem = (h0, h1)
        oxsem = (ox0, ox1)
        ossem = (os0, os1)

        def start_in(c, b):
            gh = pltpu.async_copy(
                x_hbm.at[tok_v.at[pl.ds(c * CH, CH)]], rbuf[b], gsem[b])
            sh = pltpu.async_copy(
                sb_hbm.at[pl.ds(j0 + c * CH, CH)], qbuf[b], hsem[b])
            return gh, sh

        pend = [start_in(0, 0), start_in(1, 1)]
        out_pend = [None, None]
        for c in range(nch):
            b = c % 2
            gh, sh = pend[b]
            gh.wait()
            sh.wait()
            oh = (
                pltpu.async_copy(rbuf[b], xg_hbm.at[idx_v.at[c]], oxsem[b]),
                pltpu.async_copy(qbuf[b], sp_hbm.at[idx_v.at[c]], ossem[b]),
            )
            out_pend[b] = oh
            if c + 2 < nch:
                oh[0].wait()
                oh[1].wait()
                out_pend[b] = None
                pend[b] = start_in(c + 2, b)
        for b in range(2):
            if out_pend[b] is not None:
                out_pend[b][0].wait()
                out_pend[b][1].wait()

    return permute


def _run_schedule(key_flat, src_a, src_b):
    L = key_flat.shape[0]
    first = jnp.concatenate([
        jnp.ones((1,), jnp.int32),
        (key_flat[1:] != key_flat[:-1]).astype(jnp.int32),
    ])
    run_id = jnp.cumsum(first) - 1
    rr = jnp.arange(L, dtype=jnp.int32)[:, None]
    mask = (run_id[None, :] == rr) & (first[None, :] == 1)
    def per_run(v):
        r = jnp.max(jnp.where(mask, v[None, :], -1), axis=1)
        return jnp.concatenate([r, jnp.full((1,), -1, jnp.int32)])
    a_by_run = per_run(src_a)
    b_by_run = per_run(src_b)
    nxt = jnp.clip(run_id + 1, 0, L)
    pf_a = jnp.where(first == 1, a_by_run[nxt], -1).astype(jnp.int32)
    pf_b = jnp.where(first == 1, b_by_run[nxt], -1).astype(jnp.int32)
    pf_slot = ((run_id + 1) % 2).astype(jnp.int32)
    cur_slot = (run_id % 2).astype(jnp.int32)
    k0 = (jnp.arange(L, dtype=jnp.int32) == 0)
    i2_a = jnp.where(k0, src_a[0], -1).astype(jnp.int32)
    i2_b = jnp.where(k0, src_b[0], -1).astype(jnp.int32)
    return first, cur_slot, pf_a, pf_b, pf_slot, i2_a, i2_b


def _ffn_a_body(first_r, cs_r, pfe_r, pfj_r, pfs_r, i2e_r, i2j_r,
                xg_ref, s_ref, wg_hbm, wu_hbm, h_ref,
                wgb, wub, sg, su):
    t_steps = pl.num_programs(1)
    k = pl.program_id(0) * t_steps + pl.program_id(1)
    e2 = i2e_r[k]
    j2 = i2j_r[k]

    @pl.when(e2 >= 0)
    def _():
        sl2 = pl.ds(j2 * FB, FB)
        pltpu.make_async_copy(wg_hbm.at[e2, :, sl2], wgb.at[0], sg.at[0]).start()
        pltpu.make_async_copy(wu_hbm.at[e2, :, sl2], wub.at[0], su.at[0]).start()

    pe = pfe_r[k]
    pj = pfj_r[k]
    ps = pfs_r[k]

    @pl.when(pe >= 0)
    def _():
        slp = pl.ds(pj * FB, FB)
        pltpu.make_async_copy(wg_hbm.at[pe, :, slp], wgb.at[ps], sg.at[ps]).start()
        pltpu.make_async_copy(wu_hbm.at[pe, :, slp], wub.at[ps], su.at[ps]).start()

    cs = cs_r[k]

    @pl.when(first_r[k] == 1)
    def _():
        sl0 = pl.ds(0, FB)
        pltpu.make_async_copy(wg_hbm.at[0, :, sl0], wgb.at[cs], sg.at[cs]).wait()
        pltpu.make_async_copy(wu_hbm.at[0, :, sl0], wub.at[cs], su.at[cs]).wait()

    xs = (xg_ref[...] * s_ref[:, 0:1]).astype(jnp.bfloat16)
    wg = wgb[cs].astype(jnp.bfloat16)
    wu = wub[cs].astype(jnp.bfloat16)
    g = jnp.dot(xs, wg, preferred_element_type=jnp.float32)
    u = jnp.dot(xs, wu, preferred_element_type=jnp.float32)
    h_ref[...] = (g * (1.0 / (1.0 + jnp.exp(-g))) * u).astype(jnp.bfloat16)


def _ffn_b_body(first_r, cs_r, pfe_r, pfs_r, i2e_r,
                h_ref, wd_hbm, out_ref, wdb, sd):
    k = pl.program_id(0)
    e2 = i2e_r[k]

    @pl.when(e2 >= 0)
    def _():
        pltpu.make_async_copy(wd_hbm.at[e2], wdb.at[0], sd.at[0]).start()

    pe = pfe_r[k]
    ps = pfs_r[k]

    @pl.when(pe >= 0)
    def _():
        pltpu.make_async_copy(wd_hbm.at[pe], wdb.at[ps], sd.at[ps]).start()

    cs = cs_r[k]

    @pl.when(first_r[k] == 1)
    def _():
        pltpu.make_async_copy(wd_hbm.at[0], wdb.at[cs], sd.at[cs]).wait()

    h = h_ref[...]
    wd = wdb[cs].astype(jnp.bfloat16)
    out_ref[...] = jnp.dot(h, wd, preferred_element_type=jnp.float32)


def _ffn(te, xg, s_p, w_gate, w_up, w_down, p_rows):
    t = p_rows // TM
    nj = F // FB
    jj = jnp.repeat(jnp.arange(nj, dtype=jnp.int32), t)
    ee = jnp.tile(te, nj)
    key = ee * nj + jj
    fa, csa, pfe, pfj, pfs, i2e, i2j = _run_schedule(key, ee, jj)
    h = pl.pallas_call(
        _ffn_a_body,
        grid_spec=pltpu.PrefetchScalarGridSpec(
            num_scalar_prefetch=7,
            grid=(nj, t),
            in_specs=[
                pl.BlockSpec((TM, D), lambda j, i, *_: (i, 0)),
                pl.BlockSpec((TM, LANES), lambda j, i, *_: (i, 0)),
                pl.BlockSpec(memory_space=pl.ANY),
                pl.BlockSpec(memory_space=pl.ANY),
            ],
            out_specs=pl.BlockSpec((TM, FB), lambda j, i, *_: (i, j)),
            scratch_shapes=[
                pltpu.VMEM((2, D, FB), jnp.float32),
                pltpu.VMEM((2, D, FB), jnp.float32),
                pltpu.SemaphoreType.DMA((2,)),
                pltpu.SemaphoreType.DMA((2,)),
            ],
        ),
        out_shape=jax.ShapeDtypeStruct((p_rows, F), jnp.bfloat16),
    )(fa, csa, pfe, pfj, pfs, i2e, i2j, xg, s_p, w_gate, w_up)

    fb, csb, pfe_b, _unused, pfs_b, i2e_b, _u2 = _run_schedule(
        te, te, jnp.zeros_like(te))
    return pl.pallas_call(
        _ffn_b_body,
        grid_spec=pltpu.PrefetchScalarGridSpec(
            num_scalar_prefetch=5,
            grid=(t,),
            in_specs=[
                pl.BlockSpec((TM, F), lambda i, *_: (i, 0)),
                pl.BlockSpec(memory_space=pl.ANY),
            ],
            out_specs=pl.BlockSpec((TM, D), lambda i, *_: (i, 0)),
            scratch_shapes=[
                pltpu.VMEM((2, F, D), jnp.float32),
                pltpu.SemaphoreType.DMA((2,)),
            ],
        ),
        out_shape=jax.ShapeDtypeStruct((p_rows, D), jnp.float32),
    )(fb, csb, pfe_b, pfs_b, i2e_b, h, w_down)


def _make_sc_combine(n, cc):
    tok_w = n // NW
    nch = tok_w // cc

    @functools.partial(
        pl.kernel,
        out_type=jax.ShapeDtypeStruct((n, D), jnp.float32),
        mesh=_mesh,
        scratch_types=[
            pltpu.VMEM((tok_w,), jnp.int32),
            pltpu.VMEM((tok_w,), jnp.int32),
            pltpu.VMEM((cc, D), jnp.float32),
            pltpu.VMEM((cc, D), jnp.float32),
            pltpu.VMEM((cc, D), jnp.float32),
            pltpu.VMEM((cc, D), jnp.float32),
        ]
        + [pltpu.SemaphoreType.DMA] * 6,
    )
    def combine(y_hbm, c0_hbm, c1_hbm, out_hbm, i0_v, i1_v, a0, a1, b0, b1,
                ga0, ga1, gb0, gb1, st0, st1):
        wid = lax.axis_index("s") * NC + lax.axis_index("c")
        base = wid * tok_w
        pltpu.sync_copy(c0_hbm.at[pl.ds(base, tok_w)], i0_v)
        pltpu.sync_copy(c1_hbm.at[pl.ds(base, tok_w)], i1_v)
        abuf = (a0, a1)
        bbuf = (b0, b1)
        gas = (ga0, ga1)
        gbs = (gb0, gb1)
        sts = (st0, st1)

        def start_in(c, b):
            return (
                pltpu.async_copy(y_hbm.at[i0_v.at[pl.ds(c * cc, cc)]], abuf[b], gas[b]),
                pltpu.async_copy(y_hbm.at[i1_v.at[pl.ds(c * cc, cc)]], bbuf[b], gbs[b]),
            )

        pend = [start_in(0, 0), start_in(1, 1)]
        out_pend = [None, None]
        for c in range(nch):
            b = c % 2
            pend[b][0].wait()
            pend[b][1].wait()
            a_v, b_v = abuf[b], bbuf[b]

            def add_row(r, carry):
                for k in range(D // 16):
                    sl = pl.ds(k * 16, 16)
                    a_v[r, sl] = a_v[r, sl] + b_v[r, sl]
                return carry

            lax.fori_loop(0, cc, add_row, 0)
            oh = pltpu.async_copy(a_v, out_hbm.at[pl.ds(base + c * cc, cc)], sts[b])
            out_pend[b] = oh
            if c + 2 < nch:
                oh.wait()
                out_pend[b] = None
                pend[b] = start_in(c + 2, b)
        for b in range(2):
            if out_pend[b] is not None:
                out_pend[b].wait()

    return combine


def kernel(x, gate_w, w_gate, w_up, w_down):
    bs, slen, d = x.shape
    n = bs * slen
    s_slots = n * K
    p_rows = s_slots + E * TM
    x_flat = x.reshape(n, d)

    gw_pad = jnp.zeros((d, LANES), jnp.float32).at[:, :E].set(gate_w)
    srt, idt, wnt, cnt = _router(x_flat, gw_pad, n)
    sco = srt[:, :K].reshape(-1)
    sel = idt[:, :K].reshape(-1)
    within = wnt[:, :K].reshape(-1)

    counts = cnt[0, :E]
    pc = ((counts + TM - 1) // TM) * TM
    ends = jnp.cumsum(pc)
    starts = ends - pc
    rank = (starts[sel] + within).astype(jnp.int32)
    tok = (jnp.arange(s_slots, dtype=jnp.int32) // K)
    tile_start = jnp.arange(p_rows // TM, dtype=jnp.int32) * TM
    te = jnp.sum((tile_start[:, None] >= ends[None, :]).astype(jnp.int32), axis=1)
    te = jnp.clip(te, 0, E - 1).astype(jnp.int32)

    rank3 = rank.reshape(NW, (s_slots // NW) // CH, CH)
    sb = jnp.broadcast_to(sco[:, None], (s_slots, LANES))
    xg, s_p = _make_sc_permute(s_slots, p_rows)(x_flat, rank3, tok, sb)

    y = _ffn(te, xg, s_p, w_gate, w_up, w_down, p_rows)

    cidx = rank.reshape(n, K)
    out = _make_sc_combine(n, 16)(y, cidx[:, 0], cidx[:, 1])
    return out.reshape(bs, slen, d)
